# Initial kernel scaffold; baseline (speedup 1.0000x reference)
import jax
import jax.numpy as jnp
from jax.experimental import pallas as pl


def kernel(qkvp, pfa_values, pfa_indices, rpi, rpb_table, proj_w, proj_b):
    raise NotImplementedError("write your pallas kernel here")


# speedup vs baseline: 183.9016x; 183.9016x over previous
import functools

import jax
import jax.numpy as jnp
from jax import lax
from jax.experimental import pallas as pl
from jax.experimental.pallas import tpu as pltpu
from jax.experimental.pallas import tpu_sc as plsc

B_, N, C, H, TK = 16, 256, 192, 6, 128
D = C // H
TABLE = (2 * 16 - 1) * (2 * 16 - 1)
TABLE_PAD = 1024
BH = B_ * H
ROWS = BH * N
NW = 32
ROWS_PER_W = ROWS // NW
RBATCH = 64
NBATCH = ROWS_PER_W // RBATCH
NBLK = N // NW
SCALE = D ** -0.5
EPS = 1e-20

_MESH = dict(core_axis_name="c", subcore_axis_name="s", num_cores=2,
             num_subcores=16)
_SC_PARAMS = pltpu.CompilerParams(use_tc_tiling_on_sc=False,
                                  needs_layout_passes=False)


def _wid():
    return lax.axis_index("s") * 2 + lax.axis_index("c")


@functools.partial(
    pl.kernel,
    out_type=jax.ShapeDtypeStruct((H, N, N), jnp.float32),
    mesh=plsc.VectorSubcoreMesh(**_MESH),
    scratch_types=[
        pltpu.VMEM((H, TABLE_PAD), jnp.float32),
        pltpu.VMEM((NBLK, N), jnp.int32),
        pltpu.VMEM((H, NBLK, N), jnp.float32),
    ],
    compiler_params=_SC_PARAMS,
)
def _sc_bias(tab_hbm, rpi_hbm, out_hbm, tab_buf, rpi_buf, out_buf):
    base_n = _wid() * NBLK
    pltpu.sync_copy(tab_hbm, tab_buf)
    pltpu.sync_copy(rpi_hbm.at[pl.ds(base_n, NBLK)], rpi_buf)
    for h in range(H):
        hv = jnp.full((16,), h, dtype=jnp.int32)

        def row(r, _, h=h, hv=hv):
            for j in range(N // 16):
                iv = rpi_buf[r, pl.ds(j * 16, 16)]
                out_buf[h, r, pl.ds(j * 16, 16)] = plsc.load_gather(
                    tab_buf, [hv, iv])
            return _

        lax.fori_loop(0, NBLK, row, 0)
    for h in range(H):
        pltpu.sync_copy(out_buf.at[h], out_hbm.at[h, pl.ds(base_n, NBLK), :])


def _tc_qk_body(x_ref, rpb_ref, out_ref):
    x = x_ref[0]
    for h in range(H):
        q = x[:, h * D:(h + 1) * D] * SCALE
        k = x[:, C + h * D:C + (h + 1) * D]
        a = lax.dot_general(q, k, (((1,), (1,)), ((), ())),
                            preferred_element_type=jnp.float32,
                            precision=lax.Precision.HIGHEST)
        out_ref[pl.ds(h * N, N), :] = a + rpb_ref[h]


_tc_qk = pl.pallas_call(
    _tc_qk_body,
    grid=(B_,),
    in_specs=[
        pl.BlockSpec((1, N, 4 * C), lambda b: (b, 0, 0)),
        pl.BlockSpec((H, N, N), lambda b: (0, 0, 0)),
    ],
    out_specs=pl.BlockSpec((H * N, N), lambda b: (b, 0)),
    out_shape=jax.ShapeDtypeStruct((ROWS, N), jnp.float32),
)


@functools.partial(
    pl.kernel,
    out_type=(
        jax.ShapeDtypeStruct((ROWS, TK), jnp.float32),
        jax.ShapeDtypeStruct((ROWS, N), jnp.float32),
    ),
    mesh=plsc.VectorSubcoreMesh(**_MESH),
    scratch_types=[
        pltpu.VMEM((RBATCH, N), jnp.float32),
        pltpu.VMEM((RBATCH, TK), jnp.int32),
        pltpu.VMEM((RBATCH, TK), jnp.float32),
        pltpu.VMEM((RBATCH, TK), jnp.float32),
        pltpu.VMEM((RBATCH, N), jnp.float32),
    ],
    compiler_params=_SC_PARAMS,
)
def _sc_attend(fa_hbm, idx_hbm, pfa_hbm, attn_hbm, w_hbm,
               fa_buf, idx_buf, pfa_buf, attn_buf, w_buf):
    row0 = _wid() * ROWS_PER_W

    def row(r, _):
        rv = jnp.full((16,), r, dtype=jnp.int32)
        ivs, g = [], []
        for t in range(TK // 16):
            iv = idx_buf[r, pl.ds(t * 16, 16)]
            ivs.append(iv)
            g.append(plsc.load_gather(fa_buf, [rv, iv]))
        m = g[0]
        for t in range(1, TK // 16):
            m = jnp.maximum(m, g[t])
        mv = jnp.full((16,), jnp.max(m))
        e = [jnp.exp(gt - mv) for gt in g]
        u = [et * pfa_buf[r, pl.ds(t * 16, 16)] for t, et in enumerate(e)]
        s = e[0]
        for t in range(1, TK // 16):
            s = s + e[t]
        su = u[0]
        for t in range(1, TK // 16):
            su = su + u[t]
        se = jnp.full((16,), jnp.sum(s)) * EPS
        rcp = 1.0 / (jnp.full((16,), jnp.sum(su)) + se)
        outs = []
        for t in range(TK // 16):
            o = (u[t] + se) * rcp
            outs.append(o)
            attn_buf[r, pl.ds(t * 16, 16)] = o
        for t in range(N // 16):
            w_buf[r, pl.ds(t * 16, 16)] = jnp.zeros((16,), jnp.float32)
        for t in range(TK // 16):
            plsc.addupdate_scatter(w_buf, [rv, ivs[t]], outs[t])
        return _

    for gi in range(NBATCH):
        b0 = row0 + gi * RBATCH
        pltpu.sync_copy(fa_hbm.at[pl.ds(b0, RBATCH)], fa_buf)
        pltpu.sync_copy(idx_hbm.at[pl.ds(b0, RBATCH)], idx_buf)
        pltpu.sync_copy(pfa_hbm.at[pl.ds(b0, RBATCH)], pfa_buf)
        lax.fori_loop(0, RBATCH, row, 0)
        pltpu.sync_copy(attn_buf, attn_hbm.at[pl.ds(b0, RBATCH)])
        pltpu.sync_copy(w_buf, w_hbm.at[pl.ds(b0, RBATCH)])


def _tc_av_body(w_ref, x_ref, pw_ref, pb_ref, out_ref):
    x = x_ref[0]
    outs = []
    for h in range(H):
        wm = w_ref[pl.ds(h * N, N), :]
        v = x[:, 2 * C + h * D:2 * C + (h + 1) * D]
        lep = x[:, 3 * C + h * D:3 * C + (h + 1) * D]
        outs.append(lax.dot_general(wm, v, (((1,), (0,)), ((), ())),
                                    preferred_element_type=jnp.float32,
                                    precision=lax.Precision.HIGHEST) + lep)
    cat = jnp.concatenate(outs, axis=1)
    y = lax.dot_general(cat, pw_ref[...], (((1,), (0,)), ((), ())),
                        preferred_element_type=jnp.float32,
                        precision=lax.Precision.HIGHEST) + pb_ref[...]
    out_ref[0] = y


_tc_av = pl.pallas_call(
    _tc_av_body,
    grid=(B_,),
    in_specs=[
        pl.BlockSpec((H * N, N), lambda b: (b, 0)),
        pl.BlockSpec((1, N, 4 * C), lambda b: (b, 0, 0)),
        pl.BlockSpec((C, C), lambda b: (0, 0)),
        pl.BlockSpec((1, C), lambda b: (0, 0)),
    ],
    out_specs=pl.BlockSpec((1, N, C), lambda b: (b, 0, 0)),
    out_shape=jax.ShapeDtypeStruct((B_, N, C), jnp.float32),
)


def kernel(qkvp, pfa_values, pfa_indices, rpi, rpb_table, proj_w, proj_b):
    tab = jnp.zeros((H, TABLE_PAD), jnp.float32).at[:, :TABLE].set(
        rpb_table.T.astype(jnp.float32))
    rpi32 = rpi.astype(jnp.int32)
    idx_flat = pfa_indices.astype(jnp.int32).reshape(ROWS, TK)
    pfa_flat = pfa_values.astype(jnp.float32).reshape(ROWS, TK)

    rpb_full = _sc_bias(tab, rpi32)
    fa = _tc_qk(qkvp, rpb_full)
    attn_flat, w_flat = _sc_attend(fa, idx_flat, pfa_flat)
    x = _tc_av(w_flat, qkvp, proj_w.T.astype(jnp.float32),
               proj_b.reshape(1, C).astype(jnp.float32))

    new_pfa = attn_flat.reshape(1, B_, H, N, TK)
    return x, new_pfa, pfa_indices


# speedup vs baseline: 231.4483x; 1.2585x over previous
import functools

import jax
import jax.numpy as jnp
from jax import lax
from jax.experimental import pallas as pl
from jax.experimental.pallas import tpu as pltpu
from jax.experimental.pallas import tpu_sc as plsc

B_, N, C, H, TK = 16, 256, 192, 6, 128
D = C // H
TABLE = (2 * 16 - 1) * (2 * 16 - 1)
TABLE_PAD = 1024
BH = B_ * H
ROWS = BH * N
NW = 32
ROWS_PER_W = ROWS // NW
RBATCH = 64
NBATCH = ROWS_PER_W // RBATCH
NBLK = N // NW
SCALE = D ** -0.5
EPS = 1e-20

_MESH = dict(core_axis_name="c", subcore_axis_name="s", num_cores=2,
             num_subcores=16)
_SC_PARAMS = pltpu.CompilerParams(use_tc_tiling_on_sc=False,
                                  needs_layout_passes=False)


def _wid():
    return lax.axis_index("s") * 2 + lax.axis_index("c")


@functools.partial(
    pl.kernel,
    out_type=jax.ShapeDtypeStruct((H, N, N), jnp.float32),
    mesh=plsc.VectorSubcoreMesh(**_MESH),
    scratch_types=[
        pltpu.VMEM((H, TABLE_PAD), jnp.float32),
        pltpu.VMEM((NBLK, N), jnp.int32),
        pltpu.VMEM((H, NBLK, N), jnp.float32),
    ],
    compiler_params=_SC_PARAMS,
)
def _sc_bias(tab_hbm, rpi_hbm, out_hbm, tab_buf, rpi_buf, out_buf):
    base_n = _wid() * NBLK
    pltpu.sync_copy(tab_hbm, tab_buf)
    pltpu.sync_copy(rpi_hbm.at[pl.ds(base_n, NBLK)], rpi_buf)
    for h in range(H):
        hv = jnp.full((16,), h, dtype=jnp.int32)

        def row(r, _, h=h, hv=hv):
            for j in range(N // 16):
                iv = rpi_buf[r, pl.ds(j * 16, 16)]
                out_buf[h, r, pl.ds(j * 16, 16)] = plsc.load_gather(
                    tab_buf, [hv, iv])
            return _

        lax.fori_loop(0, NBLK, row, 0)
    for h in range(H):
        pltpu.sync_copy(out_buf.at[h], out_hbm.at[h, pl.ds(base_n, NBLK), :])


def _tc_qk_body(x_ref, rpb_ref, out_ref):
    x = x_ref[0]
    for h in range(H):
        q = x[:, h * D:(h + 1) * D] * SCALE
        k = x[:, C + h * D:C + (h + 1) * D]
        a = lax.dot_general(q, k, (((1,), (1,)), ((), ())),
                            preferred_element_type=jnp.float32,
                            precision=lax.Precision.HIGHEST)
        out_ref[pl.ds(h * N, N), :] = a + rpb_ref[h]


_tc_qk = pl.pallas_call(
    _tc_qk_body,
    grid=(B_,),
    in_specs=[
        pl.BlockSpec((1, N, 4 * C), lambda b: (b, 0, 0)),
        pl.BlockSpec((H, N, N), lambda b: (0, 0, 0)),
    ],
    out_specs=pl.BlockSpec((H * N, N), lambda b: (b, 0)),
    out_shape=jax.ShapeDtypeStruct((ROWS, N), jnp.float32),
)


@functools.partial(
    pl.kernel,
    out_type=(
        jax.ShapeDtypeStruct((ROWS, TK), jnp.float32),
        jax.ShapeDtypeStruct((ROWS, N), jnp.float32),
    ),
    mesh=plsc.VectorSubcoreMesh(**_MESH),
    scratch_types=[
        pltpu.VMEM((2, RBATCH, N), jnp.float32),
        pltpu.VMEM((2, RBATCH, TK), jnp.int32),
        pltpu.VMEM((2, RBATCH, TK), jnp.float32),
        pltpu.VMEM((2, RBATCH, TK), jnp.float32),
        pltpu.VMEM((2, RBATCH, N), jnp.float32),
        pltpu.SemaphoreType.DMA,
        pltpu.SemaphoreType.DMA,
        pltpu.SemaphoreType.DMA,
        pltpu.SemaphoreType.DMA,
    ],
    compiler_params=_SC_PARAMS,
)
def _sc_attend(fa_hbm, idx_hbm, pfa_hbm, attn_hbm, w_hbm,
               fa_buf, idx_buf, pfa_buf, attn_buf, w_buf,
               sem_in0, sem_in1, sem_out0, sem_out1):
    row0 = _wid() * ROWS_PER_W
    sem_in = (sem_in0, sem_in1)
    sem_out = (sem_out0, sem_out1)

    def start_in(g):
        s = g % 2
        b0 = row0 + g * RBATCH
        return [
            pltpu.async_copy(fa_hbm.at[pl.ds(b0, RBATCH)], fa_buf.at[s],
                             sem_in[s]),
            pltpu.async_copy(idx_hbm.at[pl.ds(b0, RBATCH)], idx_buf.at[s],
                             sem_in[s]),
            pltpu.async_copy(pfa_hbm.at[pl.ds(b0, RBATCH)], pfa_buf.at[s],
                             sem_in[s]),
        ]

    def start_out(g):
        s = g % 2
        b0 = row0 + g * RBATCH
        return [
            pltpu.async_copy(attn_buf.at[s], attn_hbm.at[pl.ds(b0, RBATCH)],
                             sem_out[s]),
            pltpu.async_copy(w_buf.at[s], w_hbm.at[pl.ds(b0, RBATCH)],
                             sem_out[s]),
        ]

    def run_rows(s):
        @plsc.parallel_loop(0, RBATCH, unroll=2)
        def _(r):
            rv = jnp.full((16,), r, dtype=jnp.int32)
            ivs, g = [], []
            for t in range(TK // 16):
                iv = idx_buf[s, r, pl.ds(t * 16, 16)]
                ivs.append(iv)
                g.append(plsc.load_gather(fa_buf.at[s], [rv, iv]))
            m = g[0]
            for t in range(1, TK // 16):
                m = jnp.maximum(m, g[t])
            mv = jnp.full((16,), jnp.max(m))
            e = [jnp.exp(gt - mv) for gt in g]
            u = [et * pfa_buf[s, r, pl.ds(t * 16, 16)]
                 for t, et in enumerate(e)]
            sv = e[0]
            for t in range(1, TK // 16):
                sv = sv + e[t]
            su = u[0]
            for t in range(1, TK // 16):
                su = su + u[t]
            se = jnp.full((16,), jnp.sum(sv)) * EPS
            rcp = 1.0 / (jnp.full((16,), jnp.sum(su)) + se)
            outs = []
            for t in range(TK // 16):
                o = (u[t] + se) * rcp
                outs.append(o)
                attn_buf[s, r, pl.ds(t * 16, 16)] = o
            for t in range(N // 16):
                w_buf[s, r, pl.ds(t * 16, 16)] = jnp.zeros((16,), jnp.float32)
            for t in range(TK // 16):
                plsc.addupdate_scatter(w_buf.at[s], [rv, ivs[t]], outs[t])

    ins = [None] * NBATCH
    outs_d = [None] * NBATCH
    ins[0] = start_in(0)
    for g in range(NBATCH):
        if g + 1 < NBATCH:
            ins[g + 1] = start_in(g + 1)
        for d in ins[g]:
            d.wait()
        if g >= 2:
            for d in outs_d[g - 2]:
                d.wait()
        run_rows(g % 2)
        outs_d[g] = start_out(g)
    for g in (NBATCH - 2, NBATCH - 1):
        for d in outs_d[g]:
            d.wait()


def _tc_av_body(w_ref, x_ref, pw_ref, pb_ref, out_ref):
    x = x_ref[0]
    outs = []
    for h in range(H):
        wm = w_ref[pl.ds(h * N, N), :]
        v = x[:, 2 * C + h * D:2 * C + (h + 1) * D]
        lep = x[:, 3 * C + h * D:3 * C + (h + 1) * D]
        outs.append(lax.dot_general(wm, v, (((1,), (0,)), ((), ())),
                                    preferred_element_type=jnp.float32,
                                    precision=lax.Precision.HIGHEST) + lep)
    cat = jnp.concatenate(outs, axis=1)
    y = lax.dot_general(cat, pw_ref[...], (((1,), (0,)), ((), ())),
                        preferred_element_type=jnp.float32,
                        precision=lax.Precision.HIGHEST) + pb_ref[...]
    out_ref[0] = y


_tc_av = pl.pallas_call(
    _tc_av_body,
    grid=(B_,),
    in_specs=[
        pl.BlockSpec((H * N, N), lambda b: (b, 0)),
        pl.BlockSpec((1, N, 4 * C), lambda b: (b, 0, 0)),
        pl.BlockSpec((C, C), lambda b: (0, 0)),
        pl.BlockSpec((1, C), lambda b: (0, 0)),
    ],
    out_specs=pl.BlockSpec((1, N, C), lambda b: (b, 0, 0)),
    out_shape=jax.ShapeDtypeStruct((B_, N, C), jnp.float32),
)


def kernel(qkvp, pfa_values, pfa_indices, rpi, rpb_table, proj_w, proj_b):
    tab = jnp.zeros((H, TABLE_PAD), jnp.float32).at[:, :TABLE].set(
        rpb_table.T.astype(jnp.float32))
    rpi32 = rpi.astype(jnp.int32)
    idx_flat = pfa_indices.astype(jnp.int32).reshape(ROWS, TK)
    pfa_flat = pfa_values.astype(jnp.float32).reshape(ROWS, TK)

    rpb_full = _sc_bias(tab, rpi32)
    fa = _tc_qk(qkvp, rpb_full)
    attn_flat, w_flat = _sc_attend(fa, idx_flat, pfa_flat)
    x = _tc_av(w_flat, qkvp, proj_w.T.astype(jnp.float32),
               proj_b.reshape(1, C).astype(jnp.float32))

    new_pfa = attn_flat.reshape(1, B_, H, N, TK)
    return x, new_pfa, pfa_indices


# speedup vs baseline: 295.2926x; 1.2758x over previous
import functools

import jax
import jax.numpy as jnp
from jax import lax
from jax.experimental import pallas as pl
from jax.experimental.pallas import tpu as pltpu
from jax.experimental.pallas import tpu_sc as plsc

B_, N, C, H, TK = 16, 256, 192, 6, 128
D = C // H
TABLE = (2 * 16 - 1) * (2 * 16 - 1)
TABLE_PAD = 1024
BH = B_ * H
ROWS = BH * N
NW = 32
ROWS_PER_W = ROWS // NW
RBATCH = 64
NBATCH = ROWS_PER_W // RBATCH
NBLK = N // NW
SCALE = D ** -0.5
EPS = 1e-20

_MESH = dict(core_axis_name="c", subcore_axis_name="s", num_cores=2,
             num_subcores=16)
_SC_PARAMS = pltpu.CompilerParams(use_tc_tiling_on_sc=False,
                                  needs_layout_passes=False)


def _wid():
    return lax.axis_index("s") * 2 + lax.axis_index("c")


@functools.partial(
    pl.kernel,
    out_type=jax.ShapeDtypeStruct((H, 2, N, 128), jnp.float32),
    mesh=plsc.VectorSubcoreMesh(**_MESH),
    scratch_types=[
        pltpu.VMEM((H, TABLE_PAD), jnp.float32),
        pltpu.VMEM((NBLK, N), jnp.int32),
        pltpu.VMEM((H, 2, NBLK, 128), jnp.float32),
    ],
    compiler_params=_SC_PARAMS,
)
def _sc_bias(tab_hbm, rpi_hbm, out_hbm, tab_buf, rpi_buf, out_buf):
    base_n = _wid() * NBLK
    pltpu.sync_copy(tab_hbm, tab_buf)
    pltpu.sync_copy(rpi_hbm.at[pl.ds(base_n, NBLK)], rpi_buf)
    for h in range(H):
        hv = jnp.full((16,), h, dtype=jnp.int32)

        def row(r, _, h=h, hv=hv):
            for j in range(N // 16):
                iv = rpi_buf[r, pl.ds(j * 16, 16)]
                out_buf[h, j // 8, r, pl.ds((j % 8) * 16, 16)] = (
                    plsc.load_gather(tab_buf, [hv, iv]))
            return _

        lax.fori_loop(0, NBLK, row, 0)
    for h in range(H):
        for c2 in range(2):
            pltpu.sync_copy(out_buf.at[h, c2],
                            out_hbm.at[h, c2, pl.ds(base_n, NBLK), :])


def _tc_qk_body(x_ref, rpb_ref, out_ref):
    x = x_ref[0]
    for h in range(H):
        q = x[:, h * D:(h + 1) * D] * SCALE
        k = x[:, C + h * D:C + (h + 1) * D]
        a = lax.dot_general(q, k, (((1,), (1,)), ((), ())),
                            preferred_element_type=jnp.float32,
                            precision=lax.Precision.HIGHEST)
        for c2 in range(2):
            out_ref[c2, pl.ds(h * N, N), :] = (
                a[:, c2 * 128:(c2 + 1) * 128] + rpb_ref[h, c2])


_tc_qk = pl.pallas_call(
    _tc_qk_body,
    grid=(B_,),
    in_specs=[
        pl.BlockSpec((1, N, 4 * C), lambda b: (b, 0, 0)),
        pl.BlockSpec((H, 2, N, 128), lambda b: (0, 0, 0, 0)),
    ],
    out_specs=pl.BlockSpec((2, H * N, 128), lambda b: (0, b, 0)),
    out_shape=jax.ShapeDtypeStruct((2, ROWS, 128), jnp.float32),
)


@functools.partial(
    pl.kernel,
    out_type=(
        jax.ShapeDtypeStruct((ROWS, TK), jnp.float32),
        jax.ShapeDtypeStruct((2, ROWS, 128), jnp.float32),
    ),
    mesh=plsc.VectorSubcoreMesh(**_MESH),
    scratch_types=[
        pltpu.VMEM((2, 2, RBATCH, 128), jnp.float32),
        pltpu.VMEM((2, RBATCH, TK), jnp.int32),
        pltpu.VMEM((2, RBATCH, TK), jnp.float32),
        pltpu.VMEM((2, RBATCH, TK), jnp.float32),
        pltpu.VMEM((2, 2, RBATCH, 128), jnp.float32),
        pltpu.SemaphoreType.DMA,
        pltpu.SemaphoreType.DMA,
        pltpu.SemaphoreType.DMA,
        pltpu.SemaphoreType.DMA,
    ],
    compiler_params=_SC_PARAMS,
)
def _sc_attend(fa_hbm, idx_hbm, pfa_hbm, attn_hbm, w_hbm,
               fa_buf, idx_buf, pfa_buf, attn_buf, w_buf,
               sem_in0, sem_in1, sem_out0, sem_out1):
    row0 = _wid() * ROWS_PER_W
    sem_in = (sem_in0, sem_in1)
    sem_out = (sem_out0, sem_out1)

    def start_in(g):
        s = g % 2
        b0 = row0 + g * RBATCH
        return [
            pltpu.async_copy(fa_hbm.at[0, pl.ds(b0, RBATCH)],
                             fa_buf.at[s, 0], sem_in[s]),
            pltpu.async_copy(fa_hbm.at[1, pl.ds(b0, RBATCH)],
                             fa_buf.at[s, 1], sem_in[s]),
            pltpu.async_copy(idx_hbm.at[pl.ds(b0, RBATCH)], idx_buf.at[s],
                             sem_in[s]),
            pltpu.async_copy(pfa_hbm.at[pl.ds(b0, RBATCH)], pfa_buf.at[s],
                             sem_in[s]),
        ]

    def start_out(g):
        s = g % 2
        b0 = row0 + g * RBATCH
        return [
            pltpu.async_copy(attn_buf.at[s], attn_hbm.at[pl.ds(b0, RBATCH)],
                             sem_out[s]),
            pltpu.async_copy(w_buf.at[s, 0], w_hbm.at[0, pl.ds(b0, RBATCH)],
                             sem_out[s]),
            pltpu.async_copy(w_buf.at[s, 1], w_hbm.at[1, pl.ds(b0, RBATCH)],
                             sem_out[s]),
        ]

    def run_rows(s):
        @plsc.parallel_loop(0, RBATCH, unroll=2)
        def _(r):
            rv = jnp.full((16,), r, dtype=jnp.int32)
            chs, cls, g = [], [], []
            for t in range(TK // 16):
                iv = idx_buf[s, r, pl.ds(t * 16, 16)]
                ch = jnp.right_shift(iv, 7)
                cl = jnp.bitwise_and(iv, 127)
                chs.append(ch)
                cls.append(cl)
                g.append(plsc.load_gather(fa_buf.at[s], [ch, rv, cl]))
            m = g[0]
            for t in range(1, TK // 16):
                m = jnp.maximum(m, g[t])
            mv = jnp.full((16,), jnp.max(m))
            e = [jnp.exp(gt - mv) for gt in g]
            u = [et * pfa_buf[s, r, pl.ds(t * 16, 16)]
                 for t, et in enumerate(e)]
            sv = e[0]
            for t in range(1, TK // 16):
                sv = sv + e[t]
            su = u[0]
            for t in range(1, TK // 16):
                su = su + u[t]
            se = jnp.full((16,), jnp.sum(sv)) * EPS
            rcp = 1.0 / (jnp.full((16,), jnp.sum(su)) + se)
            outs = []
            for t in range(TK // 16):
                o = (u[t] + se) * rcp
                outs.append(o)
                attn_buf[s, r, pl.ds(t * 16, 16)] = o
            for t in range(N // 16):
                w_buf[s, t // 8, r, pl.ds((t % 8) * 16, 16)] = (
                    jnp.zeros((16,), jnp.float32))
            for t in range(TK // 16):
                plsc.addupdate_scatter(w_buf.at[s], [chs[t], rv, cls[t]],
                                       outs[t])

    ins = [None] * NBATCH
    outs_d = [None] * NBATCH
    ins[0] = start_in(0)
    for g in range(NBATCH):
        if g + 1 < NBATCH:
            ins[g + 1] = start_in(g + 1)
        for d in ins[g]:
            d.wait()
        if g >= 2:
            for d in outs_d[g - 2]:
                d.wait()
        run_rows(g % 2)
        outs_d[g] = start_out(g)
    for g in (NBATCH - 2, NBATCH - 1):
        for d in outs_d[g]:
            d.wait()


def _tc_av_body(w_ref, x_ref, pw_ref, pb_ref, out_ref):
    x = x_ref[0]
    outs = []
    for h in range(H):
        wm = jnp.concatenate(
            [w_ref[0, pl.ds(h * N, N), :], w_ref[1, pl.ds(h * N, N), :]],
            axis=1)
        v = x[:, 2 * C + h * D:2 * C + (h + 1) * D]
        lep = x[:, 3 * C + h * D:3 * C + (h + 1) * D]
        outs.append(lax.dot_general(wm, v, (((1,), (0,)), ((), ())),
                                    preferred_element_type=jnp.float32,
                                    precision=lax.Precision.HIGHEST) + lep)
    cat = jnp.concatenate(outs, axis=1)
    y = lax.dot_general(cat, pw_ref[...], (((1,), (0,)), ((), ())),
                        preferred_element_type=jnp.float32,
                        precision=lax.Precision.HIGHEST) + pb_ref[...]
    out_ref[0] = y


_tc_av = pl.pallas_call(
    _tc_av_body,
    grid=(B_,),
    in_specs=[
        pl.BlockSpec((2, H * N, 128), lambda b: (0, b, 0)),
        pl.BlockSpec((1, N, 4 * C), lambda b: (b, 0, 0)),
        pl.BlockSpec((C, C), lambda b: (0, 0)),
        pl.BlockSpec((1, C), lambda b: (0, 0)),
    ],
    out_specs=pl.BlockSpec((1, N, C), lambda b: (b, 0, 0)),
    out_shape=jax.ShapeDtypeStruct((B_, N, C), jnp.float32),
)


def kernel(qkvp, pfa_values, pfa_indices, rpi, rpb_table, proj_w, proj_b):
    tab = jnp.zeros((H, TABLE_PAD), jnp.float32).at[:, :TABLE].set(
        rpb_table.T.astype(jnp.float32))
    rpi32 = rpi.astype(jnp.int32)
    idx_flat = pfa_indices.astype(jnp.int32).reshape(ROWS, TK)
    pfa_flat = pfa_values.astype(jnp.float32).reshape(ROWS, TK)

    rpb_full = _sc_bias(tab, rpi32)
    fa = _tc_qk(qkvp, rpb_full)
    attn_flat, w_flat = _sc_attend(fa, idx_flat, pfa_flat)
    x = _tc_av(w_flat, qkvp, proj_w.T.astype(jnp.float32),
               proj_b.reshape(1, C).astype(jnp.float32))

    new_pfa = attn_flat.reshape(1, B_, H, N, TK)
    return x, new_pfa, pfa_indices


# speedup vs baseline: 343.6431x; 1.1637x over previous
import functools

import jax
import jax.numpy as jnp
from jax import lax
from jax.experimental import pallas as pl
from jax.experimental.pallas import tpu as pltpu
from jax.experimental.pallas import tpu_sc as plsc

B_, N, C, H, TK = 16, 256, 192, 6, 128
D = C // H
TABLE = (2 * 16 - 1) * (2 * 16 - 1)
TABLE_PAD = 1024
BH = B_ * H
ROWS = BH * N
NW = 32
ROWS_PER_W = ROWS // NW
RBATCH = 64
NBATCH = ROWS_PER_W // RBATCH
NBLK = N // NW
SCALE = D ** -0.5
EPS = 1e-20

_MESH = dict(core_axis_name="c", subcore_axis_name="s", num_cores=2,
             num_subcores=16)
_SC_PARAMS = pltpu.CompilerParams(use_tc_tiling_on_sc=False,
                                  needs_layout_passes=False)


def _wid():
    return lax.axis_index("s") * 2 + lax.axis_index("c")


@functools.partial(
    pl.kernel,
    out_type=jax.ShapeDtypeStruct((H, 2, N, 128), jnp.float32),
    mesh=plsc.VectorSubcoreMesh(**_MESH),
    scratch_types=[
        pltpu.VMEM((H, TABLE_PAD), jnp.float32),
        pltpu.VMEM((NBLK, N), jnp.int32),
        pltpu.VMEM((H, 2, NBLK, 128), jnp.float32),
    ],
    compiler_params=_SC_PARAMS,
)
def _sc_bias(tab_hbm, rpi_hbm, out_hbm, tab_buf, rpi_buf, out_buf):
    base_n = _wid() * NBLK
    pltpu.sync_copy(tab_hbm, tab_buf)
    pltpu.sync_copy(rpi_hbm.at[pl.ds(base_n, NBLK)], rpi_buf)
    for h in range(H):
        hv = jnp.full((16,), h, dtype=jnp.int32)

        def row(r, _, h=h, hv=hv):
            for j in range(N // 16):
                iv = rpi_buf[r, pl.ds(j * 16, 16)]
                out_buf[h, j // 8, r, pl.ds((j % 8) * 16, 16)] = (
                    plsc.load_gather(tab_buf, [hv, iv]))
            return _

        lax.fori_loop(0, NBLK, row, 0)
    for h in range(H):
        for c2 in range(2):
            pltpu.sync_copy(out_buf.at[h, c2],
                            out_hbm.at[h, c2, pl.ds(base_n, NBLK), :])


def _tc_qk_body(x_ref, rpb_ref, out_ref):
    x = x_ref[0]
    for h in range(H):
        q = x[:, h * D:(h + 1) * D] * SCALE
        k = x[:, C + h * D:C + (h + 1) * D]
        a = lax.dot_general(q, k, (((1,), (1,)), ((), ())),
                            preferred_element_type=jnp.float32,
                            precision=lax.Precision.DEFAULT)
        for c2 in range(2):
            out_ref[c2, pl.ds(h * N, N), :] = (
                a[:, c2 * 128:(c2 + 1) * 128] + rpb_ref[h, c2])


_tc_qk = pl.pallas_call(
    _tc_qk_body,
    grid=(B_,),
    in_specs=[
        pl.BlockSpec((1, N, 4 * C), lambda b: (b, 0, 0)),
        pl.BlockSpec((H, 2, N, 128), lambda b: (0, 0, 0, 0)),
    ],
    out_specs=pl.BlockSpec((2, H * N, 128), lambda b: (0, b, 0)),
    out_shape=jax.ShapeDtypeStruct((2, ROWS, 128), jnp.float32),
)


@functools.partial(
    pl.kernel,
    out_type=(
        jax.ShapeDtypeStruct((ROWS, TK), jnp.float32),
        jax.ShapeDtypeStruct((2, ROWS, 128), jnp.float32),
    ),
    mesh=plsc.VectorSubcoreMesh(**_MESH),
    scratch_types=[
        pltpu.VMEM((2, 2, RBATCH, 128), jnp.float32),
        pltpu.VMEM((2, RBATCH, TK), jnp.int32),
        pltpu.VMEM((2, RBATCH, TK), jnp.float32),
        pltpu.VMEM((2, RBATCH, TK), jnp.float32),
        pltpu.VMEM((2, 2, RBATCH, 128), jnp.float32),
        pltpu.SemaphoreType.DMA,
        pltpu.SemaphoreType.DMA,
        pltpu.SemaphoreType.DMA,
        pltpu.SemaphoreType.DMA,
    ],
    compiler_params=_SC_PARAMS,
)
def _sc_attend(fa_hbm, idx_hbm, pfa_hbm, attn_hbm, w_hbm,
               fa_buf, idx_buf, pfa_buf, attn_buf, w_buf,
               sem_in0, sem_in1, sem_out0, sem_out1):
    row0 = _wid() * ROWS_PER_W
    sem_in = (sem_in0, sem_in1)
    sem_out = (sem_out0, sem_out1)

    def start_in(g):
        s = g % 2
        b0 = row0 + g * RBATCH
        return [
            pltpu.async_copy(fa_hbm.at[0, pl.ds(b0, RBATCH)],
                             fa_buf.at[s, 0], sem_in[s]),
            pltpu.async_copy(fa_hbm.at[1, pl.ds(b0, RBATCH)],
                             fa_buf.at[s, 1], sem_in[s]),
            pltpu.async_copy(idx_hbm.at[pl.ds(b0, RBATCH)], idx_buf.at[s],
                             sem_in[s]),
            pltpu.async_copy(pfa_hbm.at[pl.ds(b0, RBATCH)], pfa_buf.at[s],
                             sem_in[s]),
        ]

    def start_out(g):
        s = g % 2
        b0 = row0 + g * RBATCH
        return [
            pltpu.async_copy(attn_buf.at[s], attn_hbm.at[pl.ds(b0, RBATCH)],
                             sem_out[s]),
            pltpu.async_copy(w_buf.at[s, 0], w_hbm.at[0, pl.ds(b0, RBATCH)],
                             sem_out[s]),
            pltpu.async_copy(w_buf.at[s, 1], w_hbm.at[1, pl.ds(b0, RBATCH)],
                             sem_out[s]),
        ]

    def run_rows(s):
        @plsc.parallel_loop(0, RBATCH, unroll=2)
        def _(r):
            rv = jnp.full((16,), r, dtype=jnp.int32)
            chs, cls, g = [], [], []
            for t in range(TK // 16):
                iv = idx_buf[s, r, pl.ds(t * 16, 16)]
                ch = jnp.right_shift(iv, 7)
                cl = jnp.bitwise_and(iv, 127)
                chs.append(ch)
                cls.append(cl)
                g.append(plsc.load_gather(fa_buf.at[s], [ch, rv, cl]))
            m = g[0]
            for t in range(1, TK // 16):
                m = jnp.maximum(m, g[t])
            mv = jnp.full((16,), jnp.max(m))
            e = [jnp.exp(gt - mv) for gt in g]
            u = [et * pfa_buf[s, r, pl.ds(t * 16, 16)]
                 for t, et in enumerate(e)]
            sv = e[0]
            for t in range(1, TK // 16):
                sv = sv + e[t]
            su = u[0]
            for t in range(1, TK // 16):
                su = su + u[t]
            se = jnp.full((16,), jnp.sum(sv)) * EPS
            rcp = 1.0 / (jnp.full((16,), jnp.sum(su)) + se)
            outs = []
            for t in range(TK // 16):
                o = (u[t] + se) * rcp
                outs.append(o)
                attn_buf[s, r, pl.ds(t * 16, 16)] = o
            for t in range(N // 16):
                w_buf[s, t // 8, r, pl.ds((t % 8) * 16, 16)] = (
                    jnp.zeros((16,), jnp.float32))
            for t in range(TK // 16):
                plsc.addupdate_scatter(w_buf.at[s], [chs[t], rv, cls[t]],
                                       outs[t])

    ins = [None] * NBATCH
    outs_d = [None] * NBATCH
    ins[0] = start_in(0)
    for g in range(NBATCH):
        if g + 1 < NBATCH:
            ins[g + 1] = start_in(g + 1)
        for d in ins[g]:
            d.wait()
        if g >= 2:
            for d in outs_d[g - 2]:
                d.wait()
        run_rows(g % 2)
        outs_d[g] = start_out(g)
    for g in (NBATCH - 2, NBATCH - 1):
        for d in outs_d[g]:
            d.wait()


def _tc_av_body(w_ref, x_ref, pw_ref, pb_ref, out_ref):
    x = x_ref[0]
    outs = []
    for h in range(H):
        wm = jnp.concatenate(
            [w_ref[0, pl.ds(h * N, N), :], w_ref[1, pl.ds(h * N, N), :]],
            axis=1)
        v = x[:, 2 * C + h * D:2 * C + (h + 1) * D]
        lep = x[:, 3 * C + h * D:3 * C + (h + 1) * D]
        outs.append(lax.dot_general(wm, v, (((1,), (0,)), ((), ())),
                                    preferred_element_type=jnp.float32,
                                    precision=lax.Precision.DEFAULT) + lep)
    cat = jnp.concatenate(outs, axis=1)
    y = lax.dot_general(cat, pw_ref[...], (((1,), (0,)), ((), ())),
                        preferred_element_type=jnp.float32,
                        precision=lax.Precision.DEFAULT) + pb_ref[...]
    out_ref[0] = y


_tc_av = pl.pallas_call(
    _tc_av_body,
    grid=(B_,),
    in_specs=[
        pl.BlockSpec((2, H * N, 128), lambda b: (0, b, 0)),
        pl.BlockSpec((1, N, 4 * C), lambda b: (b, 0, 0)),
        pl.BlockSpec((C, C), lambda b: (0, 0)),
        pl.BlockSpec((1, C), lambda b: (0, 0)),
    ],
    out_specs=pl.BlockSpec((1, N, C), lambda b: (b, 0, 0)),
    out_shape=jax.ShapeDtypeStruct((B_, N, C), jnp.float32),
)


def kernel(qkvp, pfa_values, pfa_indices, rpi, rpb_table, proj_w, proj_b):
    tab = jnp.zeros((H, TABLE_PAD), jnp.float32).at[:, :TABLE].set(
        rpb_table.T.astype(jnp.float32))
    rpi32 = rpi.astype(jnp.int32)
    idx_flat = pfa_indices.astype(jnp.int32).reshape(ROWS, TK)
    pfa_flat = pfa_values.astype(jnp.float32).reshape(ROWS, TK)

    rpb_full = _sc_bias(tab, rpi32)
    fa = _tc_qk(qkvp, rpb_full)
    attn_flat, w_flat = _sc_attend(fa, idx_flat, pfa_flat)
    x = _tc_av(w_flat, qkvp, proj_w.T.astype(jnp.float32),
               proj_b.reshape(1, C).astype(jnp.float32))

    new_pfa = attn_flat.reshape(1, B_, H, N, TK)
    return x, new_pfa, pfa_indices
